# Initial kernel scaffold; baseline (speedup 1.0000x reference)
import jax
import jax.numpy as jnp
from jax.experimental import pallas as pl


def kernel(x, edge_index, eps1, W1a, b1a, W1b, b1b, g1, be1, eps2, W2a, b2a, W2b, b2b, g2, be2, eps3, W3a, b3a, W3b, b3b, g3, be3, Wl, bl):
    raise NotImplementedError("write your pallas kernel here")


# speedup vs baseline: 4.5693x; 4.5693x over previous
import functools

import jax
import jax.numpy as jnp
from jax import lax
from jax.experimental import pallas as pl
from jax.experimental.pallas import tpu as pltpu
from jax.experimental.pallas import tpu_sc as plsc

N = 10000
E = 320000
D = 128
BN_EPS = 1e-5

NC = 2
NS = 16
NW = NC * NS
CH = 80
EPW = E // NW
NCHUNK = EPW // CH
RPS = 624
TAIL = N - NS * RPS

_sc_mesh = plsc.VectorSubcoreMesh(
    core_axis_name="c", subcore_axis_name="s", num_cores=NC, num_subcores=NS
)


@functools.partial(
    pl.kernel,
    out_type=jax.ShapeDtypeStruct((NC * N, D), jnp.float32),
    mesh=_sc_mesh,
    scratch_types=[
        pltpu.VMEM((CH,), jnp.int32),
        pltpu.VMEM((CH,), jnp.int32),
        pltpu.VMEM((CH, D), jnp.float32),
        pltpu.VMEM_SHARED((N, D), jnp.float32),
        pltpu.SemaphoreType.DMA,
    ],
)
def _sc_agg(x_hbm, src_hbm, dst_hbm, out_hbm, srcv, dstv, rows, agg, sem):
    c = lax.axis_index("c")
    s = lax.axis_index("s")
    w = c * NS + s

    def zrow(i, _):
        def zcol(j, _):
            rows[i, pl.ds(j * 16, 16)] = jnp.zeros((16,), jnp.float32)
            return 0
        return lax.fori_loop(0, D // 16, zcol, 0)

    lax.fori_loop(0, CH, zrow, 0)

    r0 = s * RPS

    def zcopy(k, _):
        pltpu.sync_copy(rows.at[pl.ds(0, 16)], agg.at[pl.ds(r0 + k * 16, 16)])
        return 0

    lax.fori_loop(0, RPS // 16, zcopy, 0)

    @pl.when(s == NS - 1)
    def _zero_tail():
        pltpu.sync_copy(rows.at[pl.ds(0, TAIL)], agg.at[pl.ds(N - TAIL, TAIL)])

    plsc.subcore_barrier()

    e0 = w * EPW

    def body(k, _):
        base = e0 + k * CH
        pltpu.sync_copy(src_hbm.at[pl.ds(base, CH)], srcv)
        pltpu.sync_copy(dst_hbm.at[pl.ds(base, CH)], dstv)
        pltpu.async_copy(x_hbm.at[srcv], rows, sem).wait()
        pltpu.sync_copy(rows, agg.at[dstv], add=True)
        return 0

    lax.fori_loop(0, NCHUNK, body, 0)

    plsc.subcore_barrier()
    pltpu.sync_copy(agg.at[pl.ds(r0, RPS)], out_hbm.at[pl.ds(c * N + r0, RPS)])

    @pl.when(s == NS - 1)
    def _out_tail():
        pltpu.sync_copy(agg.at[pl.ds(N - TAIL, TAIL)],
                        out_hbm.at[pl.ds(c * N + N - TAIL, TAIL)])


def _dot(a, b):
    return jnp.dot(a, b, preferred_element_type=jnp.float32)


def _mlp_bn(h, Wa, ba, Wb, bb, g, be):
    h = jnp.maximum(_dot(h, Wa) + ba, 0.0)
    h = jnp.maximum(_dot(h, Wb) + bb, 0.0)
    mean = jnp.mean(h, axis=0, keepdims=True)
    ctr = h - mean
    var = jnp.mean(ctr * ctr, axis=0, keepdims=True)
    return ctr * lax.rsqrt(var + BN_EPS) * g + be


def _layer_body(eps_ref, x_ref, p_ref, Wa_ref, ba_ref, Wb_ref, bb_ref,
                g_ref, be_ref, o_ref):
    h = (1.0 + eps_ref[0, 0]) * x_ref[...] + p_ref[:N, :] + p_ref[N:, :]
    o_ref[...] = jnp.maximum(
        _mlp_bn(h, Wa_ref[...], ba_ref[...], Wb_ref[...], bb_ref[...],
                g_ref[...], be_ref[...]),
        0.0,
    )


def _final_body(eps_ref, x_ref, p_ref, Wa_ref, ba_ref, Wb_ref, bb_ref,
                g_ref, be_ref, Wl_ref, bl_ref, o_ref):
    h = (1.0 + eps_ref[0, 0]) * x_ref[...] + p_ref[:N, :] + p_ref[N:, :]
    h = jnp.maximum(
        _mlp_bn(h, Wa_ref[...], ba_ref[...], Wb_ref[...], bb_ref[...],
                g_ref[...], be_ref[...]),
        0.0,
    )
    o_ref[...] = _dot(h, Wl_ref[...]) + bl_ref[...]


def _tc_call(body, n_dense):
    return pl.pallas_call(
        body,
        out_shape=jax.ShapeDtypeStruct((N, D), jnp.float32),
        in_specs=[pl.BlockSpec(memory_space=pltpu.SMEM)]
        + [pl.BlockSpec(memory_space=pltpu.VMEM)] * n_dense,
        out_specs=pl.BlockSpec(memory_space=pltpu.VMEM),
    )


_layer = _tc_call(_layer_body, 8)
_final = _tc_call(_final_body, 10)


def kernel(x, edge_index, eps1, W1a, b1a, W1b, b1b, g1, be1, eps2, W2a, b2a,
           W2b, b2b, g2, be2, eps3, W3a, b3a, W3b, b3b, g3, be3, Wl, bl):
    src = edge_index[0].astype(jnp.int32)
    dst = edge_index[1].astype(jnp.int32)
    vec = lambda v: jnp.reshape(v, (1, D))
    sca = lambda v: jnp.reshape(v, (1, 1))

    p = _sc_agg(x, src, dst)
    h = _layer(sca(eps1), x, p, W1a, vec(b1a), W1b, vec(b1b), vec(g1), vec(be1))
    p = _sc_agg(h, src, dst)
    h = _layer(sca(eps2), h, p, W2a, vec(b2a), W2b, vec(b2b), vec(g2), vec(be2))
    p = _sc_agg(h, src, dst)
    return _final(sca(eps3), h, p, W3a, vec(b3a), W3b, vec(b3b), vec(g3),
                  vec(be3), Wl, vec(bl))
